# baseline (speedup 1.0000x reference)
import functools

import jax
import jax.numpy as jnp
from jax import lax
from jax.experimental import pallas as pl
from jax.experimental.pallas import tpu as pltpu
from jax.experimental.pallas import tpu_sc as plsc

D = 64
LANES = 16
NC = 2
NS = 16
NW = NC * NS

S = 128


@functools.lru_cache(maxsize=None)
def _make_kernel(N, V):
    assert N % (NW * S) == 0
    per_w = N // NW
    n_chunks = per_w // S
    mesh = plsc.VectorSubcoreMesh(core_axis_name="c", subcore_axis_name="s")

    @functools.partial(
        pl.kernel,
        mesh=mesh,
        compiler_params=pltpu.CompilerParams(use_tc_tiling_on_sc=True),
        out_type=jax.ShapeDtypeStruct((N, D), jnp.float32),
        scratch_types=[
            pltpu.VMEM((per_w,), jnp.int32),
            pltpu.VMEM((S,), jnp.int32),
            pltpu.VMEM((S, 2 * D), jnp.float32),
            pltpu.VMEM((S, D), jnp.float32),
            pltpu.VMEM((S,), jnp.float32),
            pltpu.SemaphoreType.DMA,
        ],
    )
    def k(ids_hbm, w_hbm, sc_hbm, out_hbm,
          idx_v, hidx_v, rows_v, outb_v, scf_v, sem):
        wid = lax.axis_index("s") * NC + lax.axis_index("c")
        base = wid * per_w

        pltpu.sync_copy(ids_hbm.at[pl.ds(base, per_w)], idx_v)

        def chunk_body(c, carry):
            def half_body(i, carry2):
                ig = idx_v[pl.ds(c * S + i * LANES, LANES)]
                hidx_v[pl.ds(i * LANES, LANES)] = lax.shift_right_logical(
                    ig, 1)
                return carry2

            lax.fori_loop(0, S // LANES, half_body, 0, unroll=2)

            pltpu.async_copy(sc_hbm.at[idx_v.at[pl.ds(c * S, S)]],
                             scf_v, sem).wait()
            pltpu.async_copy(w_hbm.at[hidx_v], rows_v, sem).wait()

            def group_body(g, carry2):
                sg = jnp.maximum(scf_v[pl.ds(g * LANES, LANES)],
                                 jnp.float32(1e-8))
                nsg = -sg
                ig = idx_v[pl.ds(c * S + g * LANES, LANES)]
                hg = (ig & 1) * D
                for kk in range(LANES):
                    splat = jnp.broadcast_to(sg[kk], (LANES,))
                    nsplat = jnp.broadcast_to(nsg[kk], (LANES,))
                    r = g * LANES + kk
                    h = hg[kk]
                    for j in range(D // LANES):
                        w = rows_v[r, pl.ds(h + j * LANES, LANES)]
                        outb_v[r, pl.ds(j * LANES, LANES)] = jnp.where(
                            w < 0, nsplat, splat)
                return carry2

            lax.fori_loop(0, S // LANES, group_body, 0)

            pltpu.sync_copy(outb_v, out_hbm.at[pl.ds(base + c * S, S)])
            return carry

        lax.fori_loop(0, n_chunks, chunk_body, 0)

    return k


def kernel(input_ids, weight, scales):
    B, L = input_ids.shape
    V = weight.shape[0]
    N = B * L
    ids = input_ids.reshape(N).astype(jnp.int32)
    w2 = weight.reshape(V // 2, 2 * D)
    sc_flat = scales.reshape(-1)
    out = _make_kernel(N, V)(ids, w2, sc_flat)
    return out.reshape(B, L, D)
